# baseline (speedup 1.0000x reference)
import functools

import jax
import jax.numpy as jnp
from jax import lax
from jax.experimental import pallas as pl
from jax.experimental.pallas import tpu as pltpu
from jax.experimental.pallas import tpu_sc as plsc

_B = 16384
_D = 32
_L = 16
_NC = 2
_NS = 16
_NW = _NC * _NS
_BPW = _B // _NW
_V = 1000000
_TK = 3200

_mesh = plsc.VectorSubcoreMesh(core_axis_name="c", subcore_axis_name="s")


def _xpose_body(x_ref, o_ref):
    o_ref[...] = x_ref[...].T


def _to_row_major(tbl_t):
    grid = pl.cdiv(_V, _TK)
    return pl.pallas_call(
        _xpose_body,
        grid=(grid,),
        in_specs=[pl.BlockSpec((_D, _TK), lambda j: (0, j))],
        out_specs=pl.BlockSpec((_TK, _D), lambda j: (j, 0)),
        out_shape=jax.ShapeDtypeStruct((_V, _D), jnp.float32),
    )(tbl_t)


def _body(ub_hbm, ib_hbm, ut_hbm, it_hbm, out_hbm,
          uidx_v, iidx_v, urows_v, irows_v, out_v, sem_u, sem_i):
    wid = lax.axis_index("s") * _NC + lax.axis_index("c")
    base = wid * _BPW

    pltpu.sync_copy(ub_hbm.at[pl.ds(base, _BPW)], uidx_v)
    pltpu.sync_copy(ib_hbm.at[pl.ds(base, _BPW)], iidx_v)

    cu = pltpu.async_copy(ut_hbm.at[uidx_v], urows_v, sem_u)
    ci = pltpu.async_copy(it_hbm.at[iidx_v], irows_v, sem_i)
    cu.wait()
    ci.wait()

    lanes = lax.iota(jnp.int32, _L)
    cols = [(j + lanes) & (_D - 1) for j in range(_D)]

    def group(g, carry):
        rid = g * _L + lanes
        acc = jnp.zeros((_L,), jnp.float32)
        for j in range(_D):
            uu = plsc.load_gather(urows_v, [rid, cols[j]])
            vv = plsc.load_gather(irows_v, [rid, cols[j]])
            acc = acc + uu * vv
        out_v[pl.ds(g * _L, _L)] = acc
        return carry

    lax.fori_loop(0, _BPW // _L, group, 0)

    pltpu.sync_copy(out_v, out_hbm.at[pl.ds(base, _BPW)])


@jax.jit
def _run(user_batch, item_batch, user_table_t, item_table_t):
    ut = _to_row_major(user_table_t)
    it = _to_row_major(item_table_t)
    k = functools.partial(
        pl.kernel,
        out_type=jax.ShapeDtypeStruct((_B,), jnp.float32),
        mesh=_mesh,
        scratch_types=[
            pltpu.VMEM((_BPW,), jnp.int32),
            pltpu.VMEM((_BPW,), jnp.int32),
            pltpu.VMEM((_BPW, _D), jnp.float32),
            pltpu.VMEM((_BPW, _D), jnp.float32),
            pltpu.VMEM((_BPW,), jnp.float32),
            pltpu.SemaphoreType.DMA,
            pltpu.SemaphoreType.DMA,
        ],
        compiler_params=pltpu.CompilerParams(
            needs_layout_passes=False, use_tc_tiling_on_sc=False),
    )(_body)
    return k(user_batch, item_batch, ut, it)


def kernel(user_batch, item_batch, user_table, item_table):
    return _run(user_batch.astype(jnp.int32), item_batch.astype(jnp.int32),
                user_table.T, item_table.T)
